# baseline (speedup 1.0000x reference)
import functools

import jax
import jax.numpy as jnp
from jax import lax
from jax.experimental import pallas as pl
from jax.experimental.pallas import tpu as pltpu
from jax.experimental.pallas import tpu_sc as plsc

MARGIN = 0.1
B, V = 1024, 100000

_NC, _NS, _L = 2, 16, 16
_NW = _NC * _NS
_B_SC = 512
_B_TC = B - _B_SC
_GROWS = B // _NW
_ROWS = _B_SC // _NW
_NG = _ROWS // 8
_CL = 4992
_NCH = 20
_VPC = _CL // _L
_UNROLL = 12
_BR = 32


def _take16(v, idx):
    dn = lax.GatherDimensionNumbers(offset_dims=(), collapsed_slice_dims=(0,),
                                    start_index_map=(0,))
    return lax.gather(v, idx[:, None], dn, (1,),
                      mode=lax.GatherScatterMode.PROMISE_IN_BOUNDS)


@functools.cache
def _sc_gather_gold():
    mesh = plsc.VectorSubcoreMesh(core_axis_name="c", subcore_axis_name="s")

    @functools.partial(
        pl.kernel,
        mesh=mesh,
        out_type=jax.ShapeDtypeStruct((B * _L,), jnp.float32),
        scratch_types=[
            pltpu.VMEM((_GROWS,), jnp.int32),
            pltpu.VMEM((8, 8, 128), jnp.float32),
            pltpu.VMEM((_GROWS * _L,), jnp.float32),
            pltpu.SemaphoreType.DMA,
        ],
    )
    def body(x_hbm, gold_hbm, gs_hbm, gold_v, win_v, gs_v, sem_w):
        wid = lax.axis_index("s") * _NC + lax.axis_index("c")
        base = wid * _GROWS
        pltpu.sync_copy(gold_hbm.at[pl.ds(base, _GROWS)], gold_v)
        iota = lax.iota(jnp.int32, _L)
        zero = jnp.zeros((_L,), jnp.float32)

        for gi in range(_GROWS // 8):
            row8 = pl.multiple_of(base + gi * 8, 8)
            for r8 in range(8):
                r = gi * 8 + r8
                gvec = gold_v[pl.ds((r // _L) * _L, _L)]
                g_idx = gvec[r % _L]
                c0 = pl.multiple_of((g_idx // 128) * 128, 128)
                pltpu.make_async_copy(
                    x_hbm.at[pl.ds(row8, 8), pl.ds(c0, 128)],
                    win_v.at[r8], sem_w).start()
            for r8 in range(8):
                pltpu.make_async_copy(
                    x_hbm.at[pl.ds(row8, 8), pl.ds(0, 128)],
                    win_v.at[r8], sem_w).wait()
            for r8 in range(8):
                r = gi * 8 + r8
                gvec = gold_v[pl.ds((r // _L) * _L, _L)]
                lanevec = gvec - ((gvec >> 7) << 7)
                ksplat = jnp.full((_L,), r % _L, jnp.int32)
                lane_splat = _take16(lanevec, ksplat)
                acc = zero
                for kk in range(8):
                    wv = win_v[r8, r8, pl.ds(kk * _L, _L)]
                    acc = acc + jnp.where(lane_splat == kk * _L + iota, wv, 0.0)
                for st in (1, 2, 4, 8):
                    acc = acc + _take16(acc, iota ^ st)
                gs_v[pl.ds(r * _L, _L)] = acc

        out_off = pl.multiple_of(wid * (_GROWS * _L), 8)
        pltpu.sync_copy(gs_v, gs_hbm.at[pl.ds(out_off, _GROWS * _L)])

    return body


@functools.cache
def _sc_rank_partials():
    mesh = plsc.VectorSubcoreMesh(core_axis_name="c", subcore_axis_name="s")

    @functools.partial(
        pl.kernel,
        mesh=mesh,
        out_type=(
            jax.ShapeDtypeStruct((_B_SC * _L,), jnp.float32),
            jax.ShapeDtypeStruct((_B_SC * _L,), jnp.float32),
        ),
        scratch_types=[
            pltpu.VMEM((_ROWS * _L,), jnp.float32),
            pltpu.VMEM((8, _CL), jnp.float32),
            pltpu.VMEM((8, _CL), jnp.float32),
            pltpu.VMEM((8, 256), jnp.float32),
            pltpu.VMEM((_ROWS * _L,), jnp.float32),
            pltpu.VMEM((_ROWS * _L,), jnp.float32),
            pltpu.SemaphoreType.DMA,
            pltpu.SemaphoreType.DMA,
        ],
    )
    def body(x_hbm, gs_hbm, rout_hbm, cout_hbm,
             gsp_v, buf_a, buf_b, tail_v, racc_v, cacc_v, sem_a, sem_b):
        wid = lax.axis_index("s") * _NC + lax.axis_index("c")
        base = _B_TC + wid * _ROWS
        gs_off = pl.multiple_of(base * _L, 8)
        pltpu.sync_copy(gs_hbm.at[pl.ds(gs_off, _ROWS * _L)], gsp_v)
        zero = jnp.zeros((_L,), jnp.float32)

        def chunk_src(p, row8):
            off = pl.multiple_of(p * _CL, 128)
            return x_hbm.at[pl.ds(row8, 8), pl.ds(off, _CL)]

        def process(buf, acc, gi):
            out = []
            for r8 in range(8):
                thr = gsp_v[pl.ds((gi * 8 + r8) * _L, _L)] - MARGIN
                rv, cv = acc[2 * r8], acc[2 * r8 + 1]

                def vstep(j, a, buf=buf, thr=thr, r8=r8):
                    r0, c0 = a
                    for u in range(_UNROLL):
                        v = buf[r8, pl.ds((j * _UNROLL + u) * _L, _L)]
                        t = v - thr
                        r0 = r0 + jnp.maximum(t, 0.0)
                        c0 = c0 + jnp.where(t > 0.0, 1.0, 0.0)
                    return (r0, c0)

                rv, cv = lax.fori_loop(0, _VPC // _UNROLL, vstep, (rv, cv))
                out += [rv, cv]
            return tuple(out)

        for gi in range(_NG):
            row8 = pl.multiple_of(base + gi * 8, 8)
            pltpu.make_async_copy(chunk_src(0, row8), buf_a, sem_a).start()
            acc = (zero,) * 16

            def pair(p, acc, row8=row8, gi=gi):
                pltpu.make_async_copy(
                    chunk_src(2 * p + 1, row8), buf_b, sem_b).start()
                pltpu.make_async_copy(
                    chunk_src(0, row8), buf_a, sem_a).wait()
                acc = process(buf_a, acc, gi)

                @pl.when(p < _NCH // 2 - 1)
                def _():
                    pltpu.make_async_copy(
                        chunk_src(2 * p + 2, row8), buf_a, sem_a).start()

                pltpu.make_async_copy(
                    chunk_src(0, row8), buf_b, sem_b).wait()
                acc = process(buf_b, acc, gi)
                return acc

            acc = lax.fori_loop(0, _NCH // 2, pair, acc)

            toff = pl.multiple_of(_NCH * _CL + 0 * wid, 128)
            pltpu.sync_copy(
                x_hbm.at[pl.ds(row8, 8), pl.ds(toff, 256)], tail_v)
            acc = list(acc)
            for r8 in range(8):
                thr = gsp_v[pl.ds((gi * 8 + r8) * _L, _L)] - MARGIN
                rv, cv = acc[2 * r8], acc[2 * r8 + 1]
                for j in range((V - _NCH * _CL) // _L):
                    v = tail_v[r8, pl.ds(j * _L, _L)]
                    t = v - thr
                    rv = rv + jnp.maximum(t, 0.0)
                    cv = cv + jnp.where(t > 0.0, 1.0, 0.0)
                r = gi * 8 + r8
                racc_v[pl.ds(r * _L, _L)] = rv
                cacc_v[pl.ds(r * _L, _L)] = cv

        out_off = pl.multiple_of(wid * (_ROWS * _L), 8)
        pltpu.sync_copy(racc_v, rout_hbm.at[pl.ds(out_off, _ROWS * _L)])
        pltpu.sync_copy(cacc_v, cout_hbm.at[pl.ds(out_off, _ROWS * _L)])

    return body


def _tc_body(gs_ref, x_ref, o_ref):
    xv = x_ref[...]
    g = gs_ref[:, 0:1]
    t = xv - (g - MARGIN)
    rtot = jnp.sum(jnp.maximum(t, 0.0), axis=1, keepdims=True) - MARGIN
    cnt = jnp.sum((t > 0.0).astype(jnp.float32), axis=1, keepdims=True) - 1.0
    denom = jnp.maximum(cnt, 1.0)
    loss = jnp.where(cnt > 0.0, rtot / denom, 0.0)
    o_ref[...] = (jnp.sum(loss) / B).reshape(1, 1, 1)


def _tc_call(gs2d, x):
    grid = _B_TC // _BR
    return pl.pallas_call(
        _tc_body,
        grid=(grid,),
        in_specs=[
            pl.BlockSpec((_BR, _L), lambda i: (i, 0)),
            pl.BlockSpec((_BR, V), lambda i: (i, 0)),
        ],
        out_specs=pl.BlockSpec((1, 1, 1), lambda i: (i, 0, 0)),
        out_shape=jax.ShapeDtypeStruct((grid, 1, 1), jnp.float32),
        compiler_params=pltpu.CompilerParams(
            dimension_semantics=("parallel",),
        ),
    )(gs2d, x)


def _combine_body(tc_ref, r_ref, c_ref, o_ref):
    rtot = jnp.sum(r_ref[...], axis=1, keepdims=True) - MARGIN
    ctot = jnp.sum(c_ref[...], axis=1, keepdims=True) - 1.0
    denom = jnp.maximum(ctot, 1.0)
    loss = jnp.where(ctot > 0.0, rtot / denom, 0.0)
    o_ref[...] = (jnp.sum(tc_ref[...]) + jnp.sum(loss) / B).reshape(1, 1)


def _combine_call(tc_part, rp, cp):
    return pl.pallas_call(
        _combine_body,
        out_shape=jax.ShapeDtypeStruct((1, 1), jnp.float32),
    )(tc_part, rp, cp)


def kernel(x, gold):
    gold = gold.astype(jnp.int32)
    gs = _sc_gather_gold()(x, gold)
    rp, cp = _sc_rank_partials()(x, gs)
    tc_part = _tc_call(gs.reshape(B, _L)[:_B_TC], x)
    out = _combine_call(tc_part, rp.reshape(_B_SC, _L), cp.reshape(_B_SC, _L))
    return out.reshape(())


# speedup vs baseline: 1.0222x; 1.0222x over previous
import functools

import jax
import jax.numpy as jnp
from jax import lax
from jax.experimental import pallas as pl
from jax.experimental.pallas import tpu as pltpu
from jax.experimental.pallas import tpu_sc as plsc

MARGIN = 0.1
B, V = 1024, 100000

_NC, _NS, _L = 2, 16, 16
_NW = _NC * _NS
_B_SC = 512
_B_TC = B - _B_SC
_ROWS = _B_SC // _NW
_NG = _ROWS // 8
_CL = 4992
_NCH = 20
_VPC = _CL // _L
_UNROLL = 12
_BR = 32


def _take16(v, idx):
    dn = lax.GatherDimensionNumbers(offset_dims=(), collapsed_slice_dims=(0,),
                                    start_index_map=(0,))
    return lax.gather(v, idx[:, None], dn, (1,),
                      mode=lax.GatherScatterMode.PROMISE_IN_BOUNDS)


@functools.cache
def _sc_rank_partials():
    mesh = plsc.VectorSubcoreMesh(core_axis_name="c", subcore_axis_name="s")

    @functools.partial(
        pl.kernel,
        mesh=mesh,
        out_type=(
            jax.ShapeDtypeStruct((_B_SC * _L,), jnp.float32),
            jax.ShapeDtypeStruct((_B_SC * _L,), jnp.float32),
        ),
        scratch_types=[
            pltpu.VMEM((_ROWS,), jnp.int32),
            pltpu.VMEM((8, 8, 128), jnp.float32),
            pltpu.VMEM((8 * _L,), jnp.float32),
            pltpu.VMEM((8, _CL), jnp.float32),
            pltpu.VMEM((8, _CL), jnp.float32),
            pltpu.VMEM((8, 256), jnp.float32),
            pltpu.VMEM((_ROWS * _L,), jnp.float32),
            pltpu.VMEM((_ROWS * _L,), jnp.float32),
            pltpu.SemaphoreType.DMA,
            pltpu.SemaphoreType.DMA,
            pltpu.SemaphoreType.DMA,
        ],
    )
    def body(x_hbm, gold_hbm, rout_hbm, cout_hbm,
             gold_v, win_v, thr_v, buf_a, buf_b, tail_v, racc_v, cacc_v,
             sem_w, sem_a, sem_b):
        wid = lax.axis_index("s") * _NC + lax.axis_index("c")
        base = _B_TC + wid * _ROWS
        pltpu.sync_copy(gold_hbm.at[pl.ds(base, _ROWS)], gold_v)
        iota = lax.iota(jnp.int32, _L)
        zero = jnp.zeros((_L,), jnp.float32)

        def chunk_src(p, row8):
            off = pl.multiple_of(p * _CL, 128)
            return x_hbm.at[pl.ds(row8, 8), pl.ds(off, _CL)]

        def process(buf, acc):
            out = []
            for r8 in range(8):
                thr = thr_v[pl.ds(r8 * _L, _L)]
                rv, cv = acc[2 * r8], acc[2 * r8 + 1]

                def vstep(j, a, buf=buf, thr=thr, r8=r8):
                    r0, c0 = a
                    for u in range(_UNROLL):
                        v = buf[r8, pl.ds((j * _UNROLL + u) * _L, _L)]
                        t = v - thr
                        r0 = r0 + jnp.maximum(t, 0.0)
                        c0 = c0 + jnp.where(t > 0.0, 1.0, 0.0)
                    return (r0, c0)

                rv, cv = lax.fori_loop(0, _VPC // _UNROLL, vstep, (rv, cv))
                out += [rv, cv]
            return tuple(out)

        for gi in range(_NG):
            row8 = pl.multiple_of(base + gi * 8, 8)
            for r8 in range(8):
                r = gi * 8 + r8
                gvec = gold_v[pl.ds((r // _L) * _L, _L)]
                g_idx = gvec[r % _L]
                c0 = pl.multiple_of((g_idx // 128) * 128, 128)
                pltpu.make_async_copy(
                    x_hbm.at[pl.ds(row8, 8), pl.ds(c0, 128)],
                    win_v.at[r8], sem_w).start()
            for r8 in range(8):
                pltpu.make_async_copy(
                    x_hbm.at[pl.ds(row8, 8), pl.ds(0, 128)],
                    win_v.at[r8], sem_w).wait()
            for r8 in range(8):
                r = gi * 8 + r8
                gvec = gold_v[pl.ds((r // _L) * _L, _L)]
                lanevec = gvec - ((gvec >> 7) << 7)
                ksplat = jnp.full((_L,), r % _L, jnp.int32)
                lane_splat = _take16(lanevec, ksplat)
                acc = zero
                for kk in range(8):
                    wv = win_v[r8, r8, pl.ds(kk * _L, _L)]
                    acc = acc + jnp.where(lane_splat == kk * _L + iota, wv, 0.0)
                for st in (1, 2, 4, 8):
                    acc = acc + _take16(acc, iota ^ st)
                thr_v[pl.ds(r8 * _L, _L)] = acc - MARGIN

            pltpu.make_async_copy(chunk_src(0, row8), buf_a, sem_a).start()
            acc = (zero,) * 16

            def pair(p, acc, row8=row8):
                pltpu.make_async_copy(
                    chunk_src(2 * p + 1, row8), buf_b, sem_b).start()
                pltpu.make_async_copy(
                    chunk_src(0, row8), buf_a, sem_a).wait()
                acc = process(buf_a, acc)

                @pl.when(p < _NCH // 2 - 1)
                def _():
                    pltpu.make_async_copy(
                        chunk_src(2 * p + 2, row8), buf_a, sem_a).start()

                pltpu.make_async_copy(
                    chunk_src(0, row8), buf_b, sem_b).wait()
                acc = process(buf_b, acc)
                return acc

            acc = lax.fori_loop(0, _NCH // 2, pair, acc)

            toff = pl.multiple_of(_NCH * _CL + 0 * wid, 128)
            pltpu.sync_copy(
                x_hbm.at[pl.ds(row8, 8), pl.ds(toff, 256)], tail_v)
            acc = list(acc)
            for r8 in range(8):
                thr = thr_v[pl.ds(r8 * _L, _L)]
                rv, cv = acc[2 * r8], acc[2 * r8 + 1]
                for j in range((V - _NCH * _CL) // _L):
                    v = tail_v[r8, pl.ds(j * _L, _L)]
                    t = v - thr
                    rv = rv + jnp.maximum(t, 0.0)
                    cv = cv + jnp.where(t > 0.0, 1.0, 0.0)
                r = gi * 8 + r8
                racc_v[pl.ds(r * _L, _L)] = rv
                cacc_v[pl.ds(r * _L, _L)] = cv

        out_off = pl.multiple_of(wid * (_ROWS * _L), 8)
        pltpu.sync_copy(racc_v, rout_hbm.at[pl.ds(out_off, _ROWS * _L)])
        pltpu.sync_copy(cacc_v, cout_hbm.at[pl.ds(out_off, _ROWS * _L)])

    return body


def _tc_body(gold_ref, x_ref, o_ref):
    xv = x_ref[...]
    gold = gold_ref[...]
    col = lax.broadcasted_iota(jnp.int32, xv.shape, 1)
    g = jnp.sum(jnp.where(col == gold, xv, 0.0), axis=1, keepdims=True)
    t = xv - (g - MARGIN)
    rtot = jnp.sum(jnp.maximum(t, 0.0), axis=1, keepdims=True) - MARGIN
    cnt = jnp.sum((t > 0.0).astype(jnp.float32), axis=1, keepdims=True) - 1.0
    denom = jnp.maximum(cnt, 1.0)
    loss = jnp.where(cnt > 0.0, rtot / denom, 0.0)
    o_ref[...] = (jnp.sum(loss) / B).reshape(1, 1, 1)


def _tc_call(gold2d, x):
    grid = _B_TC // _BR
    return pl.pallas_call(
        _tc_body,
        grid=(grid,),
        in_specs=[
            pl.BlockSpec((_BR, 1), lambda i: (i, 0)),
            pl.BlockSpec((_BR, V), lambda i: (i, 0)),
        ],
        out_specs=pl.BlockSpec((1, 1, 1), lambda i: (i, 0, 0)),
        out_shape=jax.ShapeDtypeStruct((grid, 1, 1), jnp.float32),
        compiler_params=pltpu.CompilerParams(
            dimension_semantics=("parallel",),
        ),
    )(gold2d, x)


def _combine_body(tc_ref, r_ref, c_ref, o_ref):
    rtot = jnp.sum(r_ref[...], axis=1, keepdims=True) - MARGIN
    ctot = jnp.sum(c_ref[...], axis=1, keepdims=True) - 1.0
    denom = jnp.maximum(ctot, 1.0)
    loss = jnp.where(ctot > 0.0, rtot / denom, 0.0)
    o_ref[...] = (jnp.sum(tc_ref[...]) + jnp.sum(loss) / B).reshape(1, 1)


def _combine_call(tc_part, rp, cp):
    return pl.pallas_call(
        _combine_body,
        out_shape=jax.ShapeDtypeStruct((1, 1), jnp.float32),
    )(tc_part, rp, cp)


def kernel(x, gold):
    gold = gold.astype(jnp.int32)
    rp, cp = _sc_rank_partials()(x, gold)
    tc_part = _tc_call(gold.reshape(B, 1), x)
    out = _combine_call(tc_part, rp.reshape(_B_SC, _L), cp.reshape(_B_SC, _L))
    return out.reshape(())
